# Initial kernel scaffold; baseline (speedup 1.0000x reference)
import jax
import jax.numpy as jnp
from jax.experimental import pallas as pl


def kernel(x, edge_index, edge_type, num_edge_types, w0, eps0, m0w1, m0b1, m0w2, m0b2, w1, eps1, m1w1, m1b1, m1w2, m1b2):
    raise NotImplementedError("write your pallas kernel here")


# speedup vs baseline: 5.9158x; 5.9158x over previous
import functools

import jax
import jax.numpy as jnp
from jax import lax
from jax.experimental import pallas as pl
from jax.experimental.pallas import tpu as pltpu
from jax.experimental.pallas import tpu_sc as plsc

BN = 1000


def _make_gidx(E, N):
    E2 = E // 128

    def _gidx_body(src_ref, et_ref, gidx_ref):
        gidx_ref[...] = et_ref[...] * N + src_ref[...]

    return pl.pallas_call(
        _gidx_body,
        out_shape=jax.ShapeDtypeStruct((E2, 128), jnp.int32),
    )


def _premult_body(h_ref, w_ref, hrw_ref):
    R = w_ref.shape[0]
    h = h_ref[...]
    for r in range(R):
        hrw_ref[r] = h * w_ref[r]


def _make_premult(N, C, R):
    nb = N // BN
    return pl.pallas_call(
        _premult_body,
        grid=(nb,),
        in_specs=[
            pl.BlockSpec((BN, C), lambda i: (i, 0)),
            pl.BlockSpec((R, C), lambda i: (0, 0)),
        ],
        out_specs=pl.BlockSpec((R, BN, C), lambda i: (0, i, 0)),
        out_shape=jax.ShapeDtypeStruct((R, N, C), jnp.float32),
    )


def _mlp_body(eps_ref, h_ref, agg_ref, w1_ref, b1_ref, w2_ref, b2_ref,
              *rest):
    ht = (1.0 + eps_ref[0, 0]) * h_ref[...] + agg_ref[0] + agg_ref[1]
    hmid = jnp.maximum(
        jnp.dot(ht, w1_ref[...], preferred_element_type=jnp.float32)
        + b1_ref[...], 0.0)
    out = (jnp.dot(hmid, w2_ref[...], preferred_element_type=jnp.float32)
           + b2_ref[...])
    if len(rest) == 1:
        (out_ref,) = rest
        out_ref[...] = out
    else:
        wn_ref, out_ref, hrw_ref = rest
        out_ref[...] = out
        for r in range(wn_ref.shape[0]):
            hrw_ref[r] = out * wn_ref[r]


def _make_mlp(N, C, R, fuse_premult):
    nb = N // BN
    in_specs = [
        pl.BlockSpec((1, 1), lambda i: (0, 0)),
        pl.BlockSpec((BN, C), lambda i: (i, 0)),
        pl.BlockSpec((2, BN, C), lambda i: (0, i, 0)),
        pl.BlockSpec((C, C), lambda i: (0, 0)),
        pl.BlockSpec((1, C), lambda i: (0, 0)),
        pl.BlockSpec((C, C), lambda i: (0, 0)),
        pl.BlockSpec((1, C), lambda i: (0, 0)),
    ]
    out_specs = pl.BlockSpec((BN, C), lambda i: (i, 0))
    out_shape = jax.ShapeDtypeStruct((N, C), jnp.float32)
    if fuse_premult:
        in_specs.append(pl.BlockSpec((R, C), lambda i: (0, 0)))
        out_specs = [out_specs, pl.BlockSpec((R, BN, C), lambda i: (0, i, 0))]
        out_shape = [out_shape, jax.ShapeDtypeStruct((R, N, C), jnp.float32)]
    return pl.pallas_call(
        _mlp_body,
        grid=(nb,),
        in_specs=in_specs,
        out_specs=out_specs,
        out_shape=out_shape,
    )


def _make_sc_agg(N, C, E):
    NC, NS = 2, 16
    EW = E // (NC * NS)
    W = 80
    NCHUNK = EW // W
    RPS = (N // NS) // 8 * 8
    RLAST = N - (NS - 1) * RPS
    ZR = 16
    mesh = plsc.VectorSubcoreMesh(core_axis_name="c", subcore_axis_name="s")

    @functools.partial(
        pl.kernel,
        out_type=jax.ShapeDtypeStruct((NC, N, C), jnp.float32),
        mesh=mesh,
        scratch_types=[
            pltpu.VMEM((W,), jnp.int32),
            pltpu.VMEM((W,), jnp.int32),
            pltpu.VMEM((W, C), jnp.float32),
            pltpu.VMEM((ZR, C), jnp.float32),
            pltpu.VMEM_SHARED((N, C), jnp.float32),
            pltpu.SemaphoreType.DMA,
        ],
    )
    def sc_agg(hrw_hbm, gidx_hbm, dst_hbm, out_hbm,
               gi_v, di_v, rows_v, zb_v, acc_sh, sem):
        cid = lax.axis_index("c")
        sid = lax.axis_index("s")
        zeros16 = jnp.zeros((16,), jnp.float32)
        row_base = pl.multiple_of(sid * RPS, 8)

        @pl.loop(0, ZR)
        def _(r):
            @pl.loop(0, C // 16)
            def _(j):
                zb_v[r, pl.ds(j * 16, 16)] = zeros16

        @pl.loop(0, RPS // ZR)
        def _(t):
            pltpu.sync_copy(
                zb_v, acc_sh.at[pl.ds(pl.multiple_of(row_base + t * ZR, 8), ZR)])

        @pl.when(sid == NS - 1)
        def _():
            @pl.loop(0, (RLAST - RPS) // ZR)
            def _(t):
                pltpu.sync_copy(
                    zb_v,
                    acc_sh.at[pl.ds((NS - 1) * RPS + RPS + t * ZR, ZR)])

        plsc.subcore_barrier()

        base = cid * (E // NC) + sid * EW

        @pl.loop(0, NCHUNK)
        def _(k):
            off = pl.multiple_of(base + k * W, 8)
            pltpu.sync_copy(gidx_hbm.at[pl.ds(off, W)], gi_v)
            pltpu.sync_copy(dst_hbm.at[pl.ds(off, W)], di_v)
            pltpu.async_copy(hrw_hbm.at[gi_v], rows_v, sem).wait()
            pltpu.sync_copy(rows_v, acc_sh.at[di_v], add=True)

        plsc.subcore_barrier()

        @pl.when(sid != NS - 1)
        def _():
            pltpu.sync_copy(acc_sh.at[pl.ds(row_base, RPS)],
                            out_hbm.at[cid, pl.ds(row_base, RPS)])

        @pl.when(sid == NS - 1)
        def _():
            pltpu.sync_copy(acc_sh.at[pl.ds((NS - 1) * RPS, RLAST)],
                            out_hbm.at[cid, pl.ds((NS - 1) * RPS, RLAST)])

    return sc_agg


def kernel(x, edge_index, edge_type, num_edge_types,
           w0, eps0, m0w1, m0b1, m0w2, m0b2,
           w1, eps1, m1w1, m1b1, m1w2, m1b2):
    N, C = x.shape
    E = edge_type.shape[0]
    R = w0.shape[0]

    src = edge_index[0]
    dst = edge_index[1]
    gidx = _make_gidx(E, N)(
        src.reshape(E // 128, 128), edge_type.reshape(E // 128, 128)
    ).reshape(E)

    sc_agg = _make_sc_agg(N, C, E)
    mlp_fused = _make_mlp(N, C, R, fuse_premult=True)
    mlp_final = _make_mlp(N, C, R, fuse_premult=False)

    hrw0 = _make_premult(N, C, R)(x, w0)
    part0 = sc_agg(hrw0.reshape(R * N, C), gidx, dst)
    h1, hrw1 = mlp_fused(eps0.reshape(1, 1), x, part0,
                         m0w1, m0b1.reshape(1, C), m0w2, m0b2.reshape(1, C),
                         w1)
    part1 = sc_agg(hrw1.reshape(R * N, C), gidx, dst)
    out = mlp_final(eps1.reshape(1, 1), h1, part1,
                    m1w1, m1b1.reshape(1, C), m1w2, m1b2.reshape(1, C))
    return out


# speedup vs baseline: 12.2189x; 2.0655x over previous
import functools

import jax
import jax.numpy as jnp
from jax import lax
from jax.experimental import pallas as pl
from jax.experimental.pallas import tpu as pltpu
from jax.experimental.pallas import tpu_sc as plsc

BN = 1000


def _make_gidx(E, N):
    E2 = E // 128

    def _gidx_body(src_ref, et_ref, dst_ref, pk_ref):
        pk_ref[:, 0, :] = et_ref[...] * N + src_ref[...]
        pk_ref[:, 1, :] = dst_ref[...]

    return pl.pallas_call(
        _gidx_body,
        out_shape=jax.ShapeDtypeStruct((E2, 2, 128), jnp.int32),
    )


def _premult_body(h_ref, w_ref, hrw_ref):
    R = w_ref.shape[0]
    h = h_ref[...]
    for r in range(R):
        hrw_ref[r] = h * w_ref[r]


def _make_premult(N, C, R):
    nb = N // BN
    return pl.pallas_call(
        _premult_body,
        grid=(nb,),
        in_specs=[
            pl.BlockSpec((BN, C), lambda i: (i, 0)),
            pl.BlockSpec((R, C), lambda i: (0, 0)),
        ],
        out_specs=pl.BlockSpec((R, BN, C), lambda i: (0, i, 0)),
        out_shape=jax.ShapeDtypeStruct((R, N, C), jnp.float32),
    )


def _mlp_body(eps_ref, h_ref, agg_ref, w1_ref, b1_ref, w2_ref, b2_ref,
              *rest):
    ht = (1.0 + eps_ref[0, 0]) * h_ref[...] + agg_ref[0] + agg_ref[1]
    hmid = jnp.maximum(
        jnp.dot(ht, w1_ref[...], preferred_element_type=jnp.float32)
        + b1_ref[...], 0.0)
    out = (jnp.dot(hmid, w2_ref[...], preferred_element_type=jnp.float32)
           + b2_ref[...])
    if len(rest) == 1:
        (out_ref,) = rest
        out_ref[...] = out
    else:
        wn_ref, out_ref, hrw_ref = rest
        out_ref[...] = out
        for r in range(wn_ref.shape[0]):
            hrw_ref[r] = out * wn_ref[r]


def _make_mlp(N, C, R, fuse_premult):
    nb = N // BN
    in_specs = [
        pl.BlockSpec((1, 1), lambda i: (0, 0)),
        pl.BlockSpec((BN, C), lambda i: (i, 0)),
        pl.BlockSpec((2, BN, C), lambda i: (0, i, 0)),
        pl.BlockSpec((C, C), lambda i: (0, 0)),
        pl.BlockSpec((1, C), lambda i: (0, 0)),
        pl.BlockSpec((C, C), lambda i: (0, 0)),
        pl.BlockSpec((1, C), lambda i: (0, 0)),
    ]
    out_specs = pl.BlockSpec((BN, C), lambda i: (i, 0))
    out_shape = jax.ShapeDtypeStruct((N, C), jnp.float32)
    if fuse_premult:
        in_specs.append(pl.BlockSpec((R, C), lambda i: (0, 0)))
        out_specs = [out_specs, pl.BlockSpec((R, BN, C), lambda i: (0, i, 0))]
        out_shape = [out_shape, jax.ShapeDtypeStruct((R, N, C), jnp.float32)]
    return pl.pallas_call(
        _mlp_body,
        grid=(nb,),
        in_specs=in_specs,
        out_specs=out_specs,
        out_shape=out_shape,
    )


def _make_sc_agg(N, C, E):
    NC, NS = 2, 16
    NW = NC * NS
    W = 128
    NCH = E // W
    NF = NCH // NW
    NEXTRA = NCH - NF * NW
    assert NF % 2 == 0 and NF >= 4
    RPS = (N // NS) // 8 * 8
    RLAST = N - (NS - 1) * RPS
    ZR = 16
    mesh = plsc.VectorSubcoreMesh(core_axis_name="c", subcore_axis_name="s")

    @functools.partial(
        pl.kernel,
        out_type=jax.ShapeDtypeStruct((NC, N, C), jnp.float32),
        mesh=mesh,
        scratch_types=[
            pltpu.VMEM((2, W), jnp.int32),
            pltpu.VMEM((2, W), jnp.int32),
            pltpu.VMEM((W, C), jnp.float32),
            pltpu.VMEM((W, C), jnp.float32),
            pltpu.VMEM((ZR, C), jnp.float32),
            pltpu.VMEM_SHARED((N, C), jnp.float32),
            pltpu.SemaphoreType.DMA,
            pltpu.SemaphoreType.DMA,
            pltpu.SemaphoreType.DMA,
            pltpu.SemaphoreType.DMA,
        ],
    )
    def sc_agg(hrw_hbm, pk_hbm, out_hbm,
               ib0, ib1, rb0, rb1, zb_v, acc_sh,
               gsem0, gsem1, isem0, isem1):
        cid = lax.axis_index("c")
        sid = lax.axis_index("s")
        wid = cid * NS + sid
        zeros16 = jnp.zeros((16,), jnp.float32)
        row_base = pl.multiple_of(sid * RPS, 8)

        @pl.loop(0, ZR)
        def _(r):
            @pl.loop(0, C // 16)
            def _(j):
                zb_v[r, pl.ds(j * 16, 16)] = zeros16

        @pl.loop(0, RPS // ZR)
        def _(t):
            pltpu.sync_copy(
                zb_v, acc_sh.at[pl.ds(pl.multiple_of(row_base + t * ZR, 8), ZR)])

        @pl.when(sid == NS - 1)
        def _():
            @pl.loop(0, (RLAST - RPS) // ZR)
            def _(t):
                pltpu.sync_copy(
                    zb_v,
                    acc_sh.at[pl.ds((NS - 1) * RPS + RPS + t * ZR, ZR)])

        plsc.subcore_barrier()

        cbase = wid * NF

        def load_idx(ch, ib, isem):
            return pltpu.async_copy(pk_hbm.at[ch], ib, isem)

        def gather(ib, rb, gsem):
            return pltpu.async_copy(hrw_hbm.at[ib.at[0]], rb, gsem)

        def wait_gather(ib, rb, gsem):
            pltpu.make_async_copy(hrw_hbm.at[ib.at[0]], rb, gsem).wait()

        def scat(rb, ib):
            pltpu.sync_copy(rb, acc_sh.at[ib.at[1]], add=True)

        load_idx(cbase, ib0, isem0).wait()
        gather(ib0, rb0, gsem0)
        load_idx(cbase + 1, ib1, isem1)

        @pl.loop(0, (NF - 2) // 2)
        def _(g):
            k = cbase + 2 * g
            wait_gather(ib0, rb0, gsem0)
            pltpu.make_async_copy(pk_hbm.at[k + 1], ib1, isem1).wait()
            gather(ib1, rb1, gsem1)
            scat(rb0, ib0)
            load_idx(k + 2, ib0, isem0)
            wait_gather(ib1, rb1, gsem1)
            pltpu.make_async_copy(pk_hbm.at[k + 2], ib0, isem0).wait()
            gather(ib0, rb0, gsem0)
            scat(rb1, ib1)
            load_idx(k + 3, ib1, isem1)

        wait_gather(ib0, rb0, gsem0)
        pltpu.make_async_copy(pk_hbm.at[cbase + NF - 1], ib1, isem1).wait()
        gather(ib1, rb1, gsem1)
        scat(rb0, ib0)

        @pl.when(wid < NEXTRA)
        def _():
            load_idx(NW * NF + wid, ib0, isem0).wait()
            gather(ib0, rb0, gsem0)

        wait_gather(ib1, rb1, gsem1)
        scat(rb1, ib1)

        @pl.when(wid < NEXTRA)
        def _():
            wait_gather(ib0, rb0, gsem0)
            scat(rb0, ib0)

        plsc.subcore_barrier()

        @pl.when(sid != NS - 1)
        def _():
            pltpu.sync_copy(acc_sh.at[pl.ds(row_base, RPS)],
                            out_hbm.at[cid, pl.ds(row_base, RPS)])

        @pl.when(sid == NS - 1)
        def _():
            pltpu.sync_copy(acc_sh.at[pl.ds((NS - 1) * RPS, RLAST)],
                            out_hbm.at[cid, pl.ds((NS - 1) * RPS, RLAST)])

    return sc_agg


def kernel(x, edge_index, edge_type, num_edge_types,
           w0, eps0, m0w1, m0b1, m0w2, m0b2,
           w1, eps1, m1w1, m1b1, m1w2, m1b2):
    N, C = x.shape
    E = edge_type.shape[0]
    R = w0.shape[0]

    src = edge_index[0]
    dst = edge_index[1]
    pk = _make_gidx(E, N)(
        src.reshape(E // 128, 128), edge_type.reshape(E // 128, 128),
        dst.reshape(E // 128, 128))

    sc_agg = _make_sc_agg(N, C, E)
    mlp_fused = _make_mlp(N, C, R, fuse_premult=True)
    mlp_final = _make_mlp(N, C, R, fuse_premult=False)

    hrw0 = _make_premult(N, C, R)(x, w0)
    part0 = sc_agg(hrw0.reshape(R * N, C), pk)
    h1, hrw1 = mlp_fused(eps0.reshape(1, 1), x, part0,
                         m0w1, m0b1.reshape(1, C), m0w2, m0b2.reshape(1, C),
                         w1)
    part1 = sc_agg(hrw1.reshape(R * N, C), pk)
    out = mlp_final(eps1.reshape(1, 1), h1, part1,
                    m1w1, m1b1.reshape(1, C), m1w2, m1b2.reshape(1, C))
    return out


# speedup vs baseline: 12.9327x; 1.0584x over previous
import functools

import jax
import jax.numpy as jnp
from jax import lax
from jax.experimental import pallas as pl
from jax.experimental.pallas import tpu as pltpu
from jax.experimental.pallas import tpu_sc as plsc

BN = 1000


def _make_gidx(E, N):
    E2 = E // 128

    def _gidx_body(src_ref, et_ref, dst_ref, pk_ref):
        pk_ref[:, 0, :] = et_ref[...] * N + src_ref[...]
        pk_ref[:, 1, :] = dst_ref[...]

    return pl.pallas_call(
        _gidx_body,
        out_shape=jax.ShapeDtypeStruct((E2, 2, 128), jnp.int32),
    )


def _premult_body(h_ref, w_ref, hrw_ref):
    R = w_ref.shape[0]
    h = h_ref[...]
    for r in range(R):
        hrw_ref[r] = h * w_ref[r]


def _make_premult(N, C, R):
    nb = N // BN
    return pl.pallas_call(
        _premult_body,
        grid=(nb,),
        in_specs=[
            pl.BlockSpec((BN, C), lambda i: (i, 0)),
            pl.BlockSpec((R, C), lambda i: (0, 0)),
        ],
        out_specs=pl.BlockSpec((R, BN, C), lambda i: (0, i, 0)),
        out_shape=jax.ShapeDtypeStruct((R, N, C), jnp.float32),
    )


def _mlp_body(eps_ref, h_ref, agg_ref, w1_ref, b1_ref, w2_ref, b2_ref,
              *rest):
    ht = (1.0 + eps_ref[0, 0]) * h_ref[...] + agg_ref[0] + agg_ref[1]
    hmid = jnp.maximum(
        jnp.dot(ht, w1_ref[...], preferred_element_type=jnp.float32)
        + b1_ref[...], 0.0)
    out = (jnp.dot(hmid, w2_ref[...], preferred_element_type=jnp.float32)
           + b2_ref[...])
    if len(rest) == 1:
        (out_ref,) = rest
        out_ref[...] = out
    else:
        wn_ref, out_ref, hrw_ref = rest
        out_ref[...] = out
        for r in range(wn_ref.shape[0]):
            hrw_ref[r] = out * wn_ref[r]


def _make_mlp(N, C, R, fuse_premult):
    nb = N // BN
    in_specs = [
        pl.BlockSpec((1, 1), lambda i: (0, 0)),
        pl.BlockSpec((BN, C), lambda i: (i, 0)),
        pl.BlockSpec((2, BN, C), lambda i: (0, i, 0)),
        pl.BlockSpec((C, C), lambda i: (0, 0)),
        pl.BlockSpec((1, C), lambda i: (0, 0)),
        pl.BlockSpec((C, C), lambda i: (0, 0)),
        pl.BlockSpec((1, C), lambda i: (0, 0)),
    ]
    out_specs = pl.BlockSpec((BN, C), lambda i: (i, 0))
    out_shape = jax.ShapeDtypeStruct((N, C), jnp.float32)
    if fuse_premult:
        in_specs.append(pl.BlockSpec((R, C), lambda i: (0, 0)))
        out_specs = [out_specs, pl.BlockSpec((R, BN, C), lambda i: (0, i, 0))]
        out_shape = [out_shape, jax.ShapeDtypeStruct((R, N, C), jnp.float32)]
    return pl.pallas_call(
        _mlp_body,
        grid=(nb,),
        in_specs=in_specs,
        out_specs=out_specs,
        out_shape=out_shape,
    )


def _make_sc_agg(N, C, E):
    NC, NS = 2, 16
    NW = NC * NS
    W = 128
    NCH = E // W
    NF = NCH // NW
    NEXTRA = NCH - NF * NW
    assert (NF - 3) % 3 == 0 and NF >= 6
    RPS = (N // NS) // 8 * 8
    RLAST = N - (NS - 1) * RPS
    mesh = plsc.VectorSubcoreMesh(core_axis_name="c", subcore_axis_name="s")

    @functools.partial(
        pl.kernel,
        out_type=jax.ShapeDtypeStruct((NC, N, C), jnp.float32),
        mesh=mesh,
        scratch_types=[
            pltpu.VMEM((2, W), jnp.int32),
            pltpu.VMEM((2, W), jnp.int32),
            pltpu.VMEM((2, W), jnp.int32),
            pltpu.VMEM((W, C), jnp.float32),
            pltpu.VMEM((W, C), jnp.float32),
            pltpu.VMEM((W, C), jnp.float32),
            pltpu.VMEM_SHARED((N, C), jnp.float32),
            pltpu.SemaphoreType.DMA,
            pltpu.SemaphoreType.DMA,
            pltpu.SemaphoreType.DMA,
            pltpu.SemaphoreType.DMA,
            pltpu.SemaphoreType.DMA,
            pltpu.SemaphoreType.DMA,
            pltpu.SemaphoreType.DMA,
        ],
    )
    def sc_agg(hrw_hbm, pk_hbm, zeros_hbm, out_hbm,
               ib0, ib1, ib2, rb0, rb1, rb2, acc_sh,
               gsem0, gsem1, gsem2, isem0, isem1, isem2, zsem):
        cid = lax.axis_index("c")
        sid = lax.axis_index("s")
        wid = cid * NS + sid
        row_base = pl.multiple_of(sid * RPS, 8)
        last_base = (NS - 1) * RPS + RPS
        cbase = wid * NF

        ibs = (ib0, ib1, ib2)
        rbs = (rb0, rb1, rb2)
        gsems = (gsem0, gsem1, gsem2)
        isems = (isem0, isem1, isem2)

        pltpu.async_copy(zeros_hbm.at[pl.ds(row_base, RPS)],
                         acc_sh.at[pl.ds(row_base, RPS)], zsem)

        @pl.when(sid == NS - 1)
        def _():
            if RLAST != RPS:
                pltpu.async_copy(
                    zeros_hbm.at[pl.ds(last_base, RLAST - RPS)],
                    acc_sh.at[pl.ds(last_base, RLAST - RPS)], zsem)

        def load_idx(s, j):
            return pltpu.async_copy(pk_hbm.at[cbase + s], ibs[j], isems[j])

        def wait_idx(s, j):
            pltpu.make_async_copy(pk_hbm.at[cbase + s], ibs[j],
                                  isems[j]).wait()

        def gather(j):
            return pltpu.async_copy(hrw_hbm.at[ibs[j].at[0]],
                                    rbs[j], gsems[j])

        def wait_gather(j):
            pltpu.make_async_copy(hrw_hbm.at[ibs[j].at[0]],
                                  rbs[j], gsems[j]).wait()

        def scat(j):
            pltpu.sync_copy(rbs[j], acc_sh.at[ibs[j].at[1]], add=True)

        load_idx(0, 0).wait()
        load_idx(1, 1).wait()
        gather(0)
        gather(1)
        load_idx(2, 2)

        pltpu.make_async_copy(zeros_hbm.at[pl.ds(row_base, RPS)],
                              acc_sh.at[pl.ds(row_base, RPS)], zsem).wait()

        @pl.when(sid == NS - 1)
        def _():
            if RLAST != RPS:
                pltpu.make_async_copy(
                    zeros_hbm.at[pl.ds(last_base, RLAST - RPS)],
                    acc_sh.at[pl.ds(last_base, RLAST - RPS)], zsem).wait()

        plsc.subcore_barrier()

        @pl.loop(0, (NF - 3) // 3)
        def _(g):
            for j in range(3):
                s = g * 3 + j
                wait_gather(j)
                wait_idx(s + 2, (j + 2) % 3)
                gather((j + 2) % 3)
                scat(j)
                load_idx(s + 3, j)

        wait_gather(0)
        wait_idx(NF - 1, 2)
        gather(2)
        scat(0)

        @pl.when(wid < NEXTRA)
        def _():
            pltpu.async_copy(pk_hbm.at[NW * NF + wid], ib0, isem0)

        wait_gather(1)
        scat(1)

        @pl.when(wid < NEXTRA)
        def _():
            pltpu.make_async_copy(pk_hbm.at[NW * NF + wid], ib0, isem0).wait()
            pltpu.async_copy(hrw_hbm.at[ib0.at[0]], rb0, gsem0)

        wait_gather(2)
        scat(2)

        @pl.when(wid < NEXTRA)
        def _():
            pltpu.make_async_copy(hrw_hbm.at[ib0.at[0]], rb0, gsem0).wait()
            pltpu.sync_copy(rb0, acc_sh.at[ib0.at[1]], add=True)

        plsc.subcore_barrier()

        @pl.when(sid != NS - 1)
        def _():
            pltpu.sync_copy(acc_sh.at[pl.ds(row_base, RPS)],
                            out_hbm.at[cid, pl.ds(row_base, RPS)])

        @pl.when(sid == NS - 1)
        def _():
            pltpu.sync_copy(acc_sh.at[pl.ds((NS - 1) * RPS, RLAST)],
                            out_hbm.at[cid, pl.ds((NS - 1) * RPS, RLAST)])

    return sc_agg


def kernel(x, edge_index, edge_type, num_edge_types,
           w0, eps0, m0w1, m0b1, m0w2, m0b2,
           w1, eps1, m1w1, m1b1, m1w2, m1b2):
    N, C = x.shape
    E = edge_type.shape[0]
    R = w0.shape[0]

    src = edge_index[0]
    dst = edge_index[1]
    pk = _make_gidx(E, N)(
        src.reshape(E // 128, 128), edge_type.reshape(E // 128, 128),
        dst.reshape(E // 128, 128))

    sc_agg = _make_sc_agg(N, C, E)
    mlp_fused = _make_mlp(N, C, R, fuse_premult=True)
    mlp_final = _make_mlp(N, C, R, fuse_premult=False)

    zeros = jnp.zeros((N, C), jnp.float32)
    hrw0 = _make_premult(N, C, R)(x, w0)
    part0 = sc_agg(hrw0.reshape(R * N, C), pk, zeros)
    h1, hrw1 = mlp_fused(eps0.reshape(1, 1), x, part0,
                         m0w1, m0b1.reshape(1, C), m0w2, m0b2.reshape(1, C),
                         w1)
    part1 = sc_agg(hrw1.reshape(R * N, C), pk, zeros)
    out = mlp_final(eps1.reshape(1, 1), h1, part1,
                    m1w1, m1b1.reshape(1, C), m1w2, m1b2.reshape(1, C))
    return out
